# Initial kernel scaffold; baseline (speedup 1.0000x reference)
import jax
import jax.numpy as jnp
from jax.experimental import pallas as pl


def kernel(x1, edge_index1, x2, edge_index2, W1, b1, W2, b2, W3, b3, g1, be1, g2, be2, g3, be3, Wp1, bp1, Wp2, bp2, Wp3, bp3):
    raise NotImplementedError("write your pallas kernel here")


# speedup vs baseline: 3.1556x; 3.1556x over previous
import functools

import jax
import jax.numpy as jnp
from jax import lax
from jax.experimental import pallas as pl
from jax.experimental.pallas import tpu as pltpu
from jax.experimental.pallas import tpu_sc as plsc

N = 50000
E = 800000
H = 256

NPAD = 50176
E_CAP = 800768
ET = E_CAP // 16
C = 128
NCHUNK = ET // C
RPT = NPAD // 16
ZR = RPT // 4
BU = 1024
NB = NPAD // BU
EPS = 1e-5
SENT_SRC = N
SENT_DST = NPAD - 1

NRC = 13824
NBK = 4
E_CAP2 = E + 512
RPT2 = NRC // 16
NRC4 = NBK * NRC

_f32 = jnp.float32


def _sc_mesh():
    return plsc.VectorSubcoreMesh(core_axis_name="c", subcore_axis_name="s")


_SC_PARAMS = pltpu.CompilerParams(use_tc_tiling_on_sc=False)


def _zero_acc(zb, acc, sid):
    for k in range(4):
        pltpu.sync_copy(zb, acc.at[pl.ds(sid * RPT + k * ZR, ZR)])


def _sc_deg(dst12, ones_hbm, zb_hbm):
    @functools.partial(
        pl.kernel,
        mesh=_sc_mesh(),
        out_type=jax.ShapeDtypeStruct((2 * NPAD, 16), _f32),
        compiler_params=_SC_PARAMS,
        scratch_types=[
            pltpu.VMEM_SHARED((NPAD, 16), _f32),
            pltpu.VMEM((C,), jnp.int32),
            pltpu.VMEM((C, 16), _f32),
            pltpu.VMEM((ZR, 16), _f32),
        ],
    )
    def body(dst_h, ones_h, zb_h, out_h, acc, dstv, onesv, zb):
        cid = lax.axis_index("c")
        sid = lax.axis_index("s")
        pltpu.sync_copy(ones_h, onesv)
        pltpu.sync_copy(zb_h, zb)
        _zero_acc(zb, acc, sid)
        plsc.subcore_barrier()

        def chunk(j, carry):
            base = pl.multiple_of(cid * E_CAP + sid * ET + j * C, C)
            pltpu.sync_copy(dst_h.at[pl.ds(base, C)], dstv)
            pltpu.sync_copy(onesv, acc.at[dstv], add=True)
            return carry

        lax.fori_loop(0, NCHUNK, chunk, 0)
        plsc.subcore_barrier()
        src_off = pl.multiple_of(sid * RPT, 8)
        dst_off = pl.multiple_of(cid * NPAD + sid * RPT, 8)
        pltpu.sync_copy(acc.at[pl.ds(src_off, RPT)], out_h.at[pl.ds(dst_off, RPT)])

    return body(dst12, ones_hbm, zb_hbm)


def _sc_accq(q12, src12, dst12, zb_hbm):
    @functools.partial(
        pl.kernel,
        mesh=_sc_mesh(),
        out_type=jax.ShapeDtypeStruct((2 * NPAD, 16), _f32),
        compiler_params=_SC_PARAMS,
        scratch_types=[
            pltpu.VMEM_SHARED((NPAD, 16), _f32),
            pltpu.VMEM((C,), jnp.int32),
            pltpu.VMEM((C,), jnp.int32),
            pltpu.VMEM((C,), jnp.int32),
            pltpu.VMEM((C, 16), _f32),
            pltpu.VMEM((ZR, 16), _f32),
            pltpu.SemaphoreType.DMA,
        ],
    )
    def body(q_h, src_h, dst_h, zb_h, out_h, acc, srcv, srcadj, dstv, rows, zb, sem):
        cid = lax.axis_index("c")
        sid = lax.axis_index("s")
        pltpu.sync_copy(zb_h, zb)
        _zero_acc(zb, acc, sid)
        plsc.subcore_barrier()

        def chunk(j, carry):
            base = pl.multiple_of(cid * E_CAP + sid * ET + j * C, C)
            pltpu.sync_copy(src_h.at[pl.ds(base, C)], srcv)
            pltpu.sync_copy(dst_h.at[pl.ds(base, C)], dstv)
            pltpu.async_copy(q_h.at[srcv], rows, sem).wait()
            pltpu.sync_copy(rows, acc.at[dstv], add=True)
            return carry

        lax.fori_loop(0, NCHUNK, chunk, 0)
        plsc.subcore_barrier()
        src_off = pl.multiple_of(sid * RPT, 8)
        dst_off = pl.multiple_of(cid * NPAD + sid * RPT, 8)
        pltpu.sync_copy(acc.at[pl.ds(src_off, RPT)], out_h.at[pl.ds(dst_off, RPT)])

    return body(q12, src12, dst12, zb_hbm)


def _sc_rows(u_flat, srcb12, dstlb12, meta12, zb_hbm):
    @functools.partial(
        pl.kernel,
        mesh=_sc_mesh(),
        out_type=jax.ShapeDtypeStruct((4 * NRC4, 128), _f32),
        compiler_params=_SC_PARAMS,
        scratch_types=[
            pltpu.VMEM_SHARED((NRC, 128), _f32),
            pltpu.VMEM((C,), jnp.int32),
            pltpu.VMEM((C,), jnp.int32),
            pltpu.VMEM((C, 128), _f32),
            pltpu.VMEM((16,), jnp.int32),
            pltpu.SemaphoreType.DMA,
        ],
    )
    def body(u_h, srcadj_h, dstl_h, meta_h, zb_h, out_h, acc, srcv, dstv,
             rows, metav, sem):
        cid = lax.axis_index("c")
        sid = lax.axis_index("s")
        pltpu.sync_copy(meta_h, metav)

        mv = metav[...]
        for b in range(NBK):
            base = jnp.where(cid == 0, mv[b], mv[8 + b])
            nblk = jnp.where(cid == 0, mv[4 + b], mv[12 + b])
            nmine = jnp.maximum(0, (nblk - sid + 15)) // 16
            for h in range(2):
                pltpu.sync_copy(zb_h, acc.at[pl.ds(sid * RPT2, RPT2)])
                plsc.subcore_barrier()

                def chunk(k, c2):
                    eb = pl.multiple_of(
                        (2 * cid + h) * E_CAP2 + base + (sid + 16 * k) * C, C)
                    pltpu.sync_copy(srcadj_h.at[pl.ds(eb, C)], srcv)
                    eb2 = pl.multiple_of(
                        cid * E_CAP2 + base + (sid + 16 * k) * C, C)
                    pltpu.sync_copy(dstl_h.at[pl.ds(eb2, C)], dstv)
                    pltpu.async_copy(u_h.at[srcv], rows, sem).wait()
                    pltpu.sync_copy(rows, acc.at[dstv], add=True)
                    return c2

                lax.fori_loop(0, nmine, chunk, 0)
                plsc.subcore_barrier()
                src_off = pl.multiple_of(sid * RPT2, 8)
                dst_off = pl.multiple_of(
                    (cid * 2 + h) * NRC4 + b * NRC + sid * RPT2, 8)
                pltpu.sync_copy(acc.at[pl.ds(src_off, RPT2)],
                                out_h.at[pl.ds(dst_off, RPT2)])
                plsc.subcore_barrier()

    return body(u_flat, srcb12, dstlb12, meta12, zb_hbm)


def _row_valid(j):
    rows = j * BU + lax.broadcasted_iota(jnp.int32, (BU, 1), 0)
    return rows < N


def _k_pre(deg12, x12):
    def body(deg_ref, x_ref, dis_ref, q_ref):
        i = pl.program_id(0)
        rows = i * BU + lax.broadcasted_iota(jnp.int32, (BU, 1), 0)
        valid = (rows % NPAD) < N
        deg = deg_ref[:, 0:1] + 1.0
        dis = jnp.where(valid, lax.rsqrt(deg), 0.0)
        dis_ref[...] = dis
        q_ref[...] = jnp.broadcast_to(dis * x_ref[...], (BU, 16))

    return pl.pallas_call(
        body,
        grid=(2 * NB,),
        in_specs=[
            pl.BlockSpec((BU, 16), lambda i: (i, 0)),
            pl.BlockSpec((BU, 1), lambda i: (i, 0)),
        ],
        out_specs=[
            pl.BlockSpec((BU, 1), lambda i: (i, 0)),
            pl.BlockSpec((BU, 16), lambda i: (i, 0)),
        ],
        out_shape=[
            jax.ShapeDtypeStruct((2 * NPAD, 1), _f32),
            jax.ShapeDtypeStruct((2 * NPAD, 16), _f32),
        ],
    )(deg12, x12)


def _k_c(dis12, q12, accq12, W1, b1r):
    def body(dis_ref, q_ref, acc_ref, w1_ref, b1_ref, c_ref, st_ref):
        j = pl.program_id(1)
        c = dis_ref[...] * (acc_ref[:, 0:1] + q_ref[:, 0:1])
        c_ref[...] = c
        t1 = jnp.maximum(c * w1_ref[...] + b1_ref[...], 0.0)
        masked = jnp.where(_row_valid(j), t1, 0.0)
        s1 = jnp.sum(masked, axis=0, keepdims=True)
        s2 = jnp.sum(masked * masked, axis=0, keepdims=True)
        cur = jnp.concatenate([s1, s2], axis=0)[None]

        @pl.when(j == 0)
        def _():
            st_ref[...] = cur

        @pl.when(j > 0)
        def _():
            st_ref[...] += cur

    return pl.pallas_call(
        body,
        grid=(2, NB),
        in_specs=[
            pl.BlockSpec((BU, 1), lambda g, j: (g * NB + j, 0)),
            pl.BlockSpec((BU, 16), lambda g, j: (g * NB + j, 0)),
            pl.BlockSpec((BU, 16), lambda g, j: (g * NB + j, 0)),
            pl.BlockSpec((1, H), lambda g, j: (0, 0)),
            pl.BlockSpec((1, H), lambda g, j: (0, 0)),
        ],
        out_specs=[
            pl.BlockSpec((BU, 1), lambda g, j: (g * NB + j, 0)),
            pl.BlockSpec((1, 2, H), lambda g, j: (g, 0, 0)),
        ],
        out_shape=[
            jax.ShapeDtypeStruct((2 * NPAD, 1), _f32),
            jax.ShapeDtypeStruct((2, 2, H), _f32),
        ],
    )(dis12, q12, accq12, W1, b1r)


def _bn_fold(st_ref, g_ref, be_ref, W_ref):
    s = st_ref[0]
    m = s[0:1] / N
    v = s[1:2] / N - m * m
    a = g_ref[...] / jnp.sqrt(v + EPS)
    r = jnp.dot(be_ref[...] - m * a, W_ref[...],
                preferred_element_type=_f32,
                precision=lax.Precision.HIGHEST)
    return a, r


def _k_u(layer1, t_or_c, dis12, stats, gp, bep, Wn, W1=None, b1r=None):
    def body(*refs):
        if layer1:
            (c_ref, dis_ref, st_ref, g_ref, be_ref, w_ref,
             w1_ref, b1_ref, u_ref) = refs
            t = jnp.maximum(c_ref[...] * w1_ref[...] + b1_ref[...], 0.0)
        else:
            (t_ref, dis_ref, st_ref, g_ref, be_ref, w_ref, u_ref) = refs
            t = t_ref[...]
        a, r = _bn_fold(st_ref, g_ref, be_ref, w_ref)
        u = dis_ref[...] * (jnp.dot(t * a, w_ref[...],
                                    preferred_element_type=_f32,
                precision=lax.Precision.HIGHEST) + r)
        u_ref[0, 0] = u[:, :128]
        u_ref[0, 1] = u[:, 128:]

    t_spec = pl.BlockSpec((BU, 1) if layer1 else (BU, H),
                          lambda g, j: (g * NB + j, 0))
    in_specs = [
        t_spec,
        pl.BlockSpec((BU, 1), lambda g, j: (g * NB + j, 0)),
        pl.BlockSpec((1, 2, H), lambda g, j: (g, 0, 0)),
        pl.BlockSpec((1, H), lambda g, j: (0, 0)),
        pl.BlockSpec((1, H), lambda g, j: (0, 0)),
        pl.BlockSpec((H, H), lambda g, j: (0, 0)),
    ]
    args = [t_or_c, dis12, stats, gp, bep, Wn]
    if layer1:
        in_specs += [pl.BlockSpec((1, H), lambda g, j: (0, 0)),
                     pl.BlockSpec((1, H), lambda g, j: (0, 0))]
        args += [W1, b1r]
    out = pl.pallas_call(
        body,
        grid=(2, NB),
        in_specs=in_specs,
        out_specs=pl.BlockSpec((1, 2, BU, 128), lambda g, j: (g, 0, j, 0)),
        out_shape=jax.ShapeDtypeStruct((2, 2, NPAD, 128), _f32),
    )(*args)
    return out


def _k_t(acc4, u4, dis12, br):
    def body(acc_ref, u_ref, dis_ref, b_ref, t_ref, st_ref):
        j = pl.program_id(1)
        dis = dis_ref[...]
        parts = [dis * (acc_ref[0, h] + u_ref[0, h]) for h in range(2)]
        t = jnp.maximum(jnp.concatenate(parts, axis=1) + b_ref[...], 0.0)
        t_ref[...] = t
        masked = jnp.where(_row_valid(j), t, 0.0)
        s1 = jnp.sum(masked, axis=0, keepdims=True)
        s2 = jnp.sum(masked * masked, axis=0, keepdims=True)
        cur = jnp.concatenate([s1, s2], axis=0)[None]

        @pl.when(j == 0)
        def _():
            st_ref[...] = cur

        @pl.when(j > 0)
        def _():
            st_ref[...] += cur

    return pl.pallas_call(
        body,
        grid=(2, NB),
        in_specs=[
            pl.BlockSpec((1, 2, BU, 128), lambda g, j: (g, 0, j, 0)),
            pl.BlockSpec((1, 2, BU, 128), lambda g, j: (g, 0, j, 0)),
            pl.BlockSpec((BU, 1), lambda g, j: (g * NB + j, 0)),
            pl.BlockSpec((1, H), lambda g, j: (0, 0)),
        ],
        out_specs=[
            pl.BlockSpec((BU, H), lambda g, j: (g * NB + j, 0)),
            pl.BlockSpec((1, 2, H), lambda g, j: (g, 0, 0)),
        ],
        out_shape=[
            jax.ShapeDtypeStruct((2 * NPAD, H), _f32),
            jax.ShapeDtypeStruct((2, 2, H), _f32),
        ],
    )(acc4, u4, dis12, br)


def _k_zp(t12, stats3, g3r, be3r, Wp1, bp1r, Wp2, bp2r, Wp3, bp3r):
    def body(t_ref, st_ref, g_ref, be_ref, w1_ref, b1_ref, w2_ref, b2_ref,
             w3_ref, b3_ref, z_ref, p_ref):
        s = st_ref[0]
        m = s[0:1] / N
        v = s[1:2] / N - m * m
        a = g_ref[...] / jnp.sqrt(v + EPS)
        z = (t_ref[...] - m) * a + be_ref[...]
        z_ref[...] = z
        h = jnp.maximum(jnp.dot(z, w1_ref[...],
                                preferred_element_type=_f32,
                precision=lax.Precision.HIGHEST) + b1_ref[...], 0.0)
        h = jnp.maximum(jnp.dot(h, w2_ref[...],
                                preferred_element_type=_f32,
                precision=lax.Precision.HIGHEST) + b2_ref[...], 0.0)
        p_ref[...] = jnp.dot(h, w3_ref[...],
                             preferred_element_type=_f32,
                precision=lax.Precision.HIGHEST) + b3_ref[...]

    return pl.pallas_call(
        body,
        grid=(2, NB),
        in_specs=[
            pl.BlockSpec((BU, H), lambda g, j: (g * NB + j, 0)),
            pl.BlockSpec((1, 2, H), lambda g, j: (g, 0, 0)),
            pl.BlockSpec((1, H), lambda g, j: (0, 0)),
            pl.BlockSpec((1, H), lambda g, j: (0, 0)),
            pl.BlockSpec((H, 512), lambda g, j: (0, 0)),
            pl.BlockSpec((1, 512), lambda g, j: (0, 0)),
            pl.BlockSpec((512, H), lambda g, j: (0, 0)),
            pl.BlockSpec((1, H), lambda g, j: (0, 0)),
            pl.BlockSpec((H, H), lambda g, j: (0, 0)),
            pl.BlockSpec((1, H), lambda g, j: (0, 0)),
        ],
        out_specs=[
            pl.BlockSpec((BU, H), lambda g, j: (g * NB + j, 0)),
            pl.BlockSpec((BU, H), lambda g, j: (g * NB + j, 0)),
        ],
        out_shape=[
            jax.ShapeDtypeStruct((2 * NPAD, H), _f32),
            jax.ShapeDtypeStruct((2 * NPAD, H), _f32),
        ],
    )(t12, stats3, g3r, be3r, Wp1, bp1r, Wp2, bp2r, Wp3, bp3r)


def _pad_edges(v, fill):
    return jnp.pad(v, (0, E_CAP - E), constant_values=fill)


def _bucketize(src, dst):
    bkt = dst // NRC
    oh = (bkt[:, None] == jnp.arange(NBK, dtype=jnp.int32)[None, :])
    csum = jnp.cumsum(oh.astype(jnp.int32), axis=0)
    rank = jnp.take_along_axis(csum, bkt[:, None], axis=1)[:, 0] - 1
    cnt = csum[-1]
    cntp = ((cnt + (C - 1)) // C) * C
    base = jnp.concatenate(
        [jnp.zeros((1,), jnp.int32), jnp.cumsum(cntp)[:NBK - 1]])
    pos = base[bkt] + rank
    srcb = jnp.full((E_CAP2,), SENT_SRC, jnp.int32).at[pos].set(src)
    dstlb = jnp.zeros((E_CAP2,), jnp.int32).at[pos].set(dst - bkt * NRC)
    return srcb, dstlb, jnp.concatenate([base, cntp // C])


def kernel(x1, edge_index1, x2, edge_index2, W1, b1, W2, b2, W3, b3,
           g1, be1, g2, be2, g3, be3, Wp1, bp1, Wp2, bp2, Wp3, bp3):
    src12 = jnp.concatenate([_pad_edges(edge_index1[0], SENT_SRC),
                             _pad_edges(edge_index2[0], SENT_SRC)])
    dst12 = jnp.concatenate([_pad_edges(edge_index1[1], SENT_DST),
                             _pad_edges(edge_index2[1], SENT_DST)])
    x12 = jnp.concatenate([jnp.pad(x1, ((0, NPAD - N), (0, 0))),
                           jnp.pad(x2, ((0, NPAD - N), (0, 0)))])
    sb1, db1, m1 = _bucketize(edge_index1[0], edge_index1[1])
    sb2, db2, m2 = _bucketize(edge_index2[0], edge_index2[1])
    srcb12h = jnp.concatenate([sb1, sb1 + NPAD, sb2 + 2 * NPAD, sb2 + 3 * NPAD])
    dstlb12 = jnp.concatenate([db1, db2])
    meta12 = jnp.concatenate([m1, m2])
    srcq12 = jnp.concatenate([_pad_edges(edge_index1[0], SENT_SRC),
                              _pad_edges(edge_index2[0], SENT_SRC) + NPAD])

    ones128 = jnp.ones((C, 16), _f32)
    zb16 = jnp.zeros((ZR, 16), _f32)
    zb128 = jnp.zeros((RPT2, 128), _f32)

    b1r, b2r, b3r = b1[None], b2[None], b3[None]
    g1r, g2r, g3r = g1[None], g2[None], g3[None]
    be1r, be2r, be3r = be1[None], be2[None], be3[None]
    bp1r, bp2r, bp3r = bp1[None], bp2[None], bp3[None]

    deg12 = _sc_deg(dst12, ones128, zb16)
    dis12, q12 = _k_pre(deg12, x12)
    accq12 = _sc_accq(q12, srcq12, dst12, zb16)
    c12, stats1 = _k_c(dis12, q12, accq12, W1, b1r)

    u2 = _k_u(True, c12, dis12, stats1, g1r, be1r, W2, W1=W1, b1r=b1r)
    acc2 = _sc_rows(u2.reshape(4 * NPAD, 128), srcb12h, dstlb12, meta12, zb128)
    t2, stats2 = _k_t(acc2.reshape(2, 2, NRC4, 128), u2, dis12, b2r)

    u3 = _k_u(False, t2, dis12, stats2, g2r, be2r, W3)
    acc3 = _sc_rows(u3.reshape(4 * NPAD, 128), srcb12h, dstlb12, meta12, zb128)
    t3, stats3 = _k_t(acc3.reshape(2, 2, NRC4, 128), u3, dis12, b3r)

    z12, p12 = _k_zp(t3, stats3, g3r, be3r, Wp1, bp1r, Wp2, bp2r, Wp3, bp3r)

    z1 = z12[:N]
    z2 = z12[NPAD:NPAD + N]
    p1 = p12[:N]
    p2 = p12[NPAD:NPAD + N]
    return (z1, z2, p1, p2)
